# Initial kernel scaffold; baseline (speedup 1.0000x reference)
import jax
import jax.numpy as jnp
from jax.experimental import pallas as pl


def kernel(x, x_edge_index, x_batch, y, y_edge_index, y_batch, params):
    raise NotImplementedError("write your pallas kernel here")


# speedup vs baseline: 4.4742x; 4.4742x over previous
import functools

import jax
import jax.numpy as jnp
from jax import lax
from jax.experimental import pallas as pl
from jax.experimental.pallas import tpu as pltpu
from jax.experimental.pallas import tpu_sc as plsc

N = 100000
E = 1600000
NUM_GRAPHS = 64
H = 30
HP = 32
CONV_LAYERS = [4, 3, 2, 2, 1]

LANES = 16
ROWS_PER_DMA = 128
DMAS_PER_CHUNK = 8
CHUNK = ROWS_PER_DMA * DMAS_PER_CHUNK
N_TILES = 16
CHUNKS_PER_TILE = 100
E_PER_TILE = CHUNK * CHUNKS_PER_TILE
E_PAD = E_PER_TILE * N_TILES
DUMMY = N
N_ACC = 100096
ROWS_PER_TILE = N_ACC // N_TILES
ZROWS = 391


def _sc_agg_body(m0, m1, src2, dst2, out0, out1, acc, zbuf, idxs, idxd, rows, sem):
    c = lax.axis_index("c")
    s = lax.axis_index("s")

    def _zrow(i, _):
        zbuf[i, :] = jnp.zeros((LANES,), jnp.float32)
        return 0
    lax.fori_loop(0, ZROWS, _zrow, 0)
    r0 = s * ROWS_PER_TILE
    for k in range(16):
        pltpu.sync_copy(zbuf, acc.at[pl.ds(r0 + k * ZROWS, ZROWS)])
    plsc.subcore_barrier()

    def _run(mh):
        def _chunk(g, _):
            roff = s * (E_PER_TILE // ROWS_PER_DMA) + g * DMAS_PER_CHUNK
            pltpu.sync_copy(src2.at[pl.ds(roff, DMAS_PER_CHUNK)], idxs)
            pltpu.sync_copy(dst2.at[pl.ds(roff, DMAS_PER_CHUNK)], idxd)
            cps = [pltpu.async_copy(mh.at[idxs.at[j]], rows.at[j], sem)
                   for j in range(DMAS_PER_CHUNK)]
            for cp in cps:
                cp.wait()
            for j in range(DMAS_PER_CHUNK):
                pltpu.sync_copy(rows.at[j], acc.at[idxd.at[j]], add=True)
            return 0
        lax.fori_loop(0, CHUNKS_PER_TILE, _chunk, 0)

    @pl.when(c == 0)
    def _():
        _run(m0)

    @pl.when(c == 1)
    def _():
        _run(m1)

    plsc.subcore_barrier()

    @pl.when(c == 0)
    def _():
        pltpu.sync_copy(acc.at[pl.ds(r0, ROWS_PER_TILE)],
                        out0.at[pl.ds(r0, ROWS_PER_TILE)])

    @pl.when(c == 1)
    def _():
        pltpu.sync_copy(acc.at[pl.ds(r0, ROWS_PER_TILE)],
                        out1.at[pl.ds(r0, ROWS_PER_TILE)])


_sc_agg = pl.kernel(
    _sc_agg_body,
    out_type=[jax.ShapeDtypeStruct((N_ACC, LANES), jnp.float32),
              jax.ShapeDtypeStruct((N_ACC, LANES), jnp.float32)],
    mesh=plsc.VectorSubcoreMesh(core_axis_name="c", subcore_axis_name="s"),
    scratch_types=[
        pltpu.VMEM_SHARED((N_ACC, LANES), jnp.float32),
        pltpu.VMEM((ZROWS, LANES), jnp.float32),
        pltpu.VMEM((DMAS_PER_CHUNK, ROWS_PER_DMA), jnp.int32),
        pltpu.VMEM((DMAS_PER_CHUNK, ROWS_PER_DMA), jnp.int32),
        pltpu.VMEM((DMAS_PER_CHUNK, ROWS_PER_DMA, LANES), jnp.float32),
        pltpu.SemaphoreType.DMA,
    ],
    compiler_params=pltpu.CompilerParams(use_tc_tiling_on_sc=False),
)


def _prep_edges(edge_index):
    src = edge_index[0].astype(jnp.int32)
    dst = edge_index[1].astype(jnp.int32)
    pad = jnp.full((E_PAD - E,), DUMMY, jnp.int32)
    src2 = jnp.concatenate([src, pad]).reshape(-1, ROWS_PER_DMA)
    dst2 = jnp.concatenate([dst, pad]).reshape(-1, ROWS_PER_DMA)
    return src2, dst2


def _aggregate(m, src2, dst2):
    mp = jnp.zeros((N_ACC, HP), jnp.float32).at[:N, :H].set(m)
    a0, a1 = _sc_agg(mp[:, :LANES], mp[:, LANES:], src2, dst2)
    return jnp.concatenate([a0, a1], axis=1)[:N, :H]


def _gru_cell(inp, h, p):
    gi = inp @ p['W_ih'].T + p['b_ih']
    gh = h @ p['W_hh'].T + p['b_hh']
    i_r, i_z, i_n = jnp.split(gi, 3, axis=-1)
    h_r, h_z, h_n = jnp.split(gh, 3, axis=-1)
    r = jax.nn.sigmoid(i_r + h_r)
    z = jax.nn.sigmoid(i_z + h_z)
    n = jnp.tanh(i_n + r * h_n)
    return (1.0 - z) * n + z * h


def _gated_conv(x, src2, dst2, p):
    L = p['weight'].shape[0]
    if x.shape[-1] < H:
        x = jnp.pad(x, ((0, 0), (0, H - x.shape[-1])))
    h = x
    for i in range(L):
        m = h @ p['weight'][i]
        agg = _aggregate(m, src2, dst2)
        h = _gru_cell(agg, h, p)
    return h


def _graph_norm(x, batch, p):
    ones = jnp.ones((x.shape[0],), dtype=x.dtype)
    counts = jnp.maximum(jax.ops.segment_sum(ones, batch, num_segments=NUM_GRAPHS), 1.0)
    mean = jax.ops.segment_sum(x, batch, num_segments=NUM_GRAPHS) / counts[:, None]
    out = x - mean[batch] * p['mean_scale']
    var = jax.ops.segment_sum(out * out, batch, num_segments=NUM_GRAPHS) / counts[:, None]
    std = jnp.sqrt(var + 1e-5)
    return p['weight'] * out / std[batch] + p['bias']


def _global_mean_pool(x, batch):
    ones = jnp.ones((x.shape[0],), dtype=x.dtype)
    counts = jnp.maximum(jax.ops.segment_sum(ones, batch, num_segments=NUM_GRAPHS), 1.0)
    return jax.ops.segment_sum(x, batch, num_segments=NUM_GRAPHS) / counts[:, None]


def _ggnn_forward(x, edge_index, batch, p):
    src2, dst2 = _prep_edges(edge_index)
    o = _gated_conv(x, src2, dst2, p['convs'][0])
    o = _graph_norm(o, batch, p['gns'][0])
    o = jax.nn.leaky_relu(o)
    o = _gated_conv(o, src2, dst2, p['convs'][1])
    o = jax.nn.leaky_relu(o)
    o = _graph_norm(o, batch, p['gns'][1])
    o = _gated_conv(o, src2, dst2, p['convs'][2])
    o = jax.nn.leaky_relu(o)
    o = _graph_norm(o, batch, p['gns'][2])
    o = _gated_conv(o, src2, dst2, p['convs'][3])
    o = jax.nn.leaky_relu(o)
    o = _graph_norm(o, batch, p['gns'][3])
    o = _gated_conv(o, src2, dst2, p['convs'][4])
    o = jax.nn.leaky_relu(o)
    return _global_mean_pool(o, batch)


def _mlp_forward(x, layers):
    n = len(layers)
    for i, l in enumerate(layers):
        x = x @ l['W'].T + l['b']
        if i + 1 < n:
            x = jax.nn.relu(x)
    return x


def kernel(x, x_edge_index, x_batch, y, y_edge_index, y_batch, params):
    msg_x = _ggnn_forward(x, x_edge_index, x_batch, params['model_a'])
    msg_y = _ggnn_forward(y, y_edge_index, y_batch, params['model_b'])
    return _mlp_forward(msg_x - msg_y, params['mlp'])
